# baseline (speedup 1.0000x reference)
import functools

import jax
import jax.numpy as jnp
from jax.experimental import pallas as pl
from jax.experimental.pallas import tpu as pltpu
from jax.experimental.pallas import tpu_sc as plsc

T, D, E, R, FF = 8192, 768, 64, 32, 768
TB1 = 2048
TB2 = 256
NUM_BLOCKS = 96
NUM_SLOTS = NUM_BLOCKS * TB2
C2 = 512
NW = 32
TPW = T // NW
HP = 128


def _gate_kernel(x_ref, wg_ref, bg_ref, g_ref, ws_ref, bs_ref, chosen_ref, h_ref):
    xb = x_ref[...]
    lg = jax.lax.dot_general(xb, wg_ref[...], (((1,), (1,)), ((), ())))
    z = g_ref[...] + (lg + bg_ref[...])
    chosen_ref[...] = jnp.argmax(z, axis=-1).astype(jnp.int32)
    h = jax.lax.dot_general(xb, ws_ref[...], (((1,), (1,)), ((), ())))
    h = jnp.maximum(h + bs_ref[...], 0.0)
    h_ref[...] = jnp.concatenate(
        [h, jnp.zeros((TB1, HP - R), jnp.float32)], axis=1)


def _gate(x, Wg, bg, G, Ws, bs):
    return pl.pallas_call(
        _gate_kernel,
        grid=(T // TB1,),
        in_specs=[
            pl.BlockSpec((TB1, D), lambda b: (b, 0)),
            pl.BlockSpec((E, D), lambda b: (0, 0)),
            pl.BlockSpec((E,), lambda b: (0,)),
            pl.BlockSpec((TB1, E), lambda b: (b, 0)),
            pl.BlockSpec((R, D), lambda b: (0, 0)),
            pl.BlockSpec((R,), lambda b: (0,)),
        ],
        out_specs=[
            pl.BlockSpec((TB1,), lambda b: (b,)),
            pl.BlockSpec((TB1, HP), lambda b: (b, 0)),
        ],
        out_shape=[
            jax.ShapeDtypeStruct((T,), jnp.int32),
            jax.ShapeDtypeStruct((T, HP), jnp.float32),
        ],
    )(x, Wg, bg, G, Ws, bs)


def _bookkeep_kernel(ch_ref, invp_ref, blk_e_ref, used_ref):
    nch = T // C2
    eio = jax.lax.broadcasted_iota(jnp.int32, (C2, E), 1)

    def count_body(i, counts):
        ch = ch_ref[pl.ds(i * C2, C2), :]
        onehot = (ch == eio).astype(jnp.float32)
        return counts + jnp.sum(onehot, axis=0, keepdims=True)

    counts = jax.lax.fori_loop(
        0, nch, count_body, jnp.zeros((1, E), jnp.float32))

    counts_i = counts.astype(jnp.int32)
    padded_i = ((counts_i + (TB2 - 1)) // TB2) * TB2
    padded_f = padded_i.astype(jnp.float32)

    mtri = (jax.lax.broadcasted_iota(jnp.int32, (E, E), 0)
            < jax.lax.broadcasted_iota(jnp.int32, (E, E), 1)).astype(jnp.float32)
    offsets = jnp.dot(padded_f, mtri, precision=jax.lax.Precision.HIGHEST)

    used = (jnp.sum(padded_i) // TB2).astype(jnp.int32)

    ltri = (jax.lax.broadcasted_iota(jnp.int32, (C2, C2), 1)
            < jax.lax.broadcasted_iota(jnp.int32, (C2, C2), 0)).astype(jnp.float32)

    def rank_body(i, running):
        ch = ch_ref[pl.ds(i * C2, C2), :]
        onehot = (ch == eio).astype(jnp.float32)
        excl = jnp.dot(ltri, onehot)
        slot = jnp.sum((offsets + running + excl) * onehot,
                       axis=1, keepdims=True)
        invp_ref[pl.ds(i * C2, C2), :] = slot.astype(jnp.int32)
        return running + jnp.sum(onehot, axis=0, keepdims=True)

    jax.lax.fori_loop(0, nch, rank_body, jnp.zeros((1, E), jnp.float32))

    bs_e = (offsets.astype(jnp.int32) // TB2)
    bc_e = padded_i // TB2
    biota = jax.lax.broadcasted_iota(jnp.int32, (NUM_BLOCKS, E), 0)
    eio_b = jax.lax.broadcasted_iota(jnp.int32, (NUM_BLOCKS, E), 1)
    within = (biota >= bs_e) & (biota < bs_e + bc_e)
    dense = jnp.sum(jnp.where(within, eio_b, 0), axis=1, keepdims=True)
    last_e = jnp.max(jnp.where(bc_e > 0, eio_b[:1, :], -1))
    blk_i = jax.lax.broadcasted_iota(jnp.int32, (NUM_BLOCKS, 1), 0)
    blk_e_ref[...] = jnp.where(blk_i < used, dense, last_e)
    used_ref[...] = jnp.full((1, 1), used, jnp.int32)


def _bookkeep(chosen2d):
    return pl.pallas_call(
        _bookkeep_kernel,
        in_specs=[pl.BlockSpec((T, 1), lambda: (0, 0))],
        out_specs=[
            pl.BlockSpec((T, 1), lambda: (0, 0)),
            pl.BlockSpec((NUM_BLOCKS, 1), lambda: (0, 0)),
            pl.BlockSpec((1, 1), lambda: (0, 0)),
        ],
        out_shape=[
            jax.ShapeDtypeStruct((T, 1), jnp.int32),
            jax.ShapeDtypeStruct((NUM_BLOCKS, 1), jnp.int32),
            jax.ShapeDtypeStruct((1, 1), jnp.int32),
        ],
    )(chosen2d)


@functools.lru_cache(maxsize=None)
def _build_scatter_h():
    mesh = plsc.VectorSubcoreMesh(core_axis_name="c", subcore_axis_name="s")

    @functools.partial(
        pl.kernel,
        mesh=mesh,
        out_type=jax.ShapeDtypeStruct((NUM_SLOTS, HP), jnp.float32),
        scratch_types=[
            pltpu.VMEM((TPW // 128, 128), jnp.int32),
            pltpu.VMEM((TPW, HP), jnp.float32),
            pltpu.SemaphoreType.DMA,
        ],
        compiler_params=pltpu.CompilerParams(use_tc_tiling_on_sc=True),
    )
    def sck(h_hbm, invp_hbm, hs_hbm, idx_v, rows_v, sem):
        wid = jax.lax.axis_index("s") * 2 + jax.lax.axis_index("c")
        base = wid * TPW
        pltpu.sync_copy(h_hbm.at[pl.ds(base, TPW)], rows_v)
        pltpu.sync_copy(invp_hbm.at[pl.ds(wid * (TPW // 128), TPW // 128)], idx_v)
        for j in range(TPW // 128):
            pltpu.async_copy(
                rows_v.at[pl.ds(j * 128, 128)], hs_hbm.at[idx_v.at[j]], sem
            ).wait()

    return sck


def _scatter_h(h, invp2d):
    return _build_scatter_h()(h, invp2d)


def _ffn_kernel(be_ref, us_ref, h_ref, w1_ref, b1_ref, w2_ref, b2_ref, o_ref):
    @pl.when(pl.program_id(0) < us_ref[0])
    def _():
        h = h_ref[...][:, :R].astype(jnp.bfloat16)
        w1 = w1_ref[0].astype(jnp.bfloat16)
        y1 = jax.lax.dot_general(h, w1, (((1,), (1,)), ((), ())),
                                 preferred_element_type=jnp.float32)
        y1 = jnp.maximum(y1 + b1_ref[0], 0.0)
        w2 = w2_ref[0].astype(jnp.bfloat16)
        y = jax.lax.dot_general(y1.astype(jnp.bfloat16), w2,
                                (((1,), (1,)), ((), ())),
                                preferred_element_type=jnp.float32)
        o_ref[...] = y + b2_ref[0]


def _ffn(blk_e, used, hs, W1s, b1s, W2s, b2s):
    grid_spec = pltpu.PrefetchScalarGridSpec(
        num_scalar_prefetch=2,
        grid=(NUM_BLOCKS,),
        in_specs=[
            pl.BlockSpec((TB2, HP), lambda b, be, us: (b, 0)),
            pl.BlockSpec((1, FF, R), lambda b, be, us: (be[b], 0, 0)),
            pl.BlockSpec((1, 1, FF), lambda b, be, us: (be[b], 0, 0)),
            pl.BlockSpec((1, D, FF), lambda b, be, us: (be[b], 0, 0)),
            pl.BlockSpec((1, 1, D), lambda b, be, us: (be[b], 0, 0)),
        ],
        out_specs=pl.BlockSpec((TB2, D), lambda b, be, us: (b, 0)),
    )
    return pl.pallas_call(
        _ffn_kernel,
        grid_spec=grid_spec,
        out_shape=jax.ShapeDtypeStruct((NUM_SLOTS, D), jnp.float32),
        compiler_params=pltpu.CompilerParams(
            dimension_semantics=("arbitrary",)),
    )(blk_e, used, hs, W1s, b1s, W2s, b2s)


@functools.lru_cache(maxsize=None)
def _build_gather_out():
    mesh = plsc.VectorSubcoreMesh(core_axis_name="c", subcore_axis_name="s")

    @functools.partial(
        pl.kernel,
        mesh=mesh,
        out_type=jax.ShapeDtypeStruct((T, D), jnp.float32),
        scratch_types=[
            pltpu.VMEM((TPW // 128, 128), jnp.int32),
            pltpu.VMEM((128, D), jnp.float32),
            pltpu.SemaphoreType.DMA,
        ],
        compiler_params=pltpu.CompilerParams(use_tc_tiling_on_sc=True),
    )
    def sck(ys_hbm, invp_hbm, out_hbm, idx_v, rows_v, sem):
        wid = jax.lax.axis_index("s") * 2 + jax.lax.axis_index("c")
        base = wid * TPW
        pltpu.sync_copy(invp_hbm.at[pl.ds(wid * (TPW // 128), TPW // 128)], idx_v)
        for j in range(TPW // 128):
            pltpu.async_copy(ys_hbm.at[idx_v.at[j]], rows_v, sem).wait()
            pltpu.sync_copy(rows_v, out_hbm.at[pl.ds(base + j * 128, 128)])

    return sck


def _gather_out(ys, invp2d):
    return _build_gather_out()(ys, invp2d)


def kernel(x, Wg, bg, Ws, bs, W1s, b1s, W2s, b2s):
    G = jax.random.gumbel(jax.random.key(42), (T, E), jnp.float32)
    chosen, h = _gate(x, Wg, bg, G, Ws, bs)
    invp, blk_e, used = _bookkeep(chosen.reshape(T, 1))
    invp2d = invp.reshape(T // 128, 128)
    hs = _scatter_h(h, invp2d)
    ys = _ffn(blk_e.reshape(NUM_BLOCKS), used.reshape(1), hs,
              W1s, b1s.reshape(E, 1, FF), W2s, b2s.reshape(E, 1, D))
    return _gather_out(ys, invp2d)


# speedup vs baseline: 1.0199x; 1.0199x over previous
import functools

import jax
import jax.numpy as jnp
from jax.experimental import pallas as pl
from jax.experimental.pallas import tpu as pltpu
from jax.experimental.pallas import tpu_sc as plsc

T, D, E, R, FF = 8192, 768, 64, 32, 768
TB1 = 2048
TB2 = 256
NUM_BLOCKS = 96
NUM_SLOTS = NUM_BLOCKS * TB2
C2 = 512
NW = 32
TPW = T // NW
HP = 128


def _gate_kernel(x_ref, wgs_ref, bg_ref, g_ref, bs_ref, chosen_ref, h_ref):
    xb = x_ref[...]
    lgh = jax.lax.dot_general(xb, wgs_ref[...], (((1,), (1,)), ((), ())))
    z = g_ref[...] + (lgh[:, :E] + bg_ref[...])
    chosen_ref[...] = jnp.argmax(z, axis=-1).astype(jnp.int32)
    h = jnp.maximum(lgh[:, E:E + R] + bs_ref[...], 0.0)
    h_ref[...] = jnp.concatenate(
        [h, jnp.zeros((TB1, HP - R), jnp.float32)], axis=1)


def _gate(x, Wg, bg, G, Ws, bs):
    wgs = jnp.concatenate([Wg, Ws], axis=0)
    return pl.pallas_call(
        _gate_kernel,
        grid=(T // TB1,),
        in_specs=[
            pl.BlockSpec((TB1, D), lambda b: (b, 0)),
            pl.BlockSpec((E + R, D), lambda b: (0, 0)),
            pl.BlockSpec((E,), lambda b: (0,)),
            pl.BlockSpec((TB1, E), lambda b: (b, 0)),
            pl.BlockSpec((R,), lambda b: (0,)),
        ],
        out_specs=[
            pl.BlockSpec((TB1,), lambda b: (b,)),
            pl.BlockSpec((TB1, HP), lambda b: (b, 0)),
        ],
        out_shape=[
            jax.ShapeDtypeStruct((T,), jnp.int32),
            jax.ShapeDtypeStruct((T, HP), jnp.float32),
        ],
    )(x, wgs, bg, G, bs)


def _bookkeep_kernel(ch_ref, invp_ref, blk_e_ref, used_ref):
    nch = T // C2
    eio = jax.lax.broadcasted_iota(jnp.int32, (C2, E), 1)

    def count_body(i, counts):
        ch = ch_ref[pl.ds(i * C2, C2), :]
        onehot = (ch == eio).astype(jnp.float32)
        return counts + jnp.sum(onehot, axis=0, keepdims=True)

    counts = jax.lax.fori_loop(
        0, nch, count_body, jnp.zeros((1, E), jnp.float32))

    counts_i = counts.astype(jnp.int32)
    padded_i = ((counts_i + (TB2 - 1)) // TB2) * TB2
    padded_f = padded_i.astype(jnp.float32)

    mtri = (jax.lax.broadcasted_iota(jnp.int32, (E, E), 0)
            < jax.lax.broadcasted_iota(jnp.int32, (E, E), 1)).astype(jnp.float32)
    offsets = jnp.dot(padded_f, mtri, precision=jax.lax.Precision.HIGHEST)

    used = (jnp.sum(padded_i) // TB2).astype(jnp.int32)

    ltri = (jax.lax.broadcasted_iota(jnp.int32, (C2, C2), 1)
            < jax.lax.broadcasted_iota(jnp.int32, (C2, C2), 0)).astype(jnp.float32)

    def rank_body(i, running):
        ch = ch_ref[pl.ds(i * C2, C2), :]
        onehot = (ch == eio).astype(jnp.float32)
        excl = jnp.dot(ltri, onehot)
        slot = jnp.sum((offsets + running + excl) * onehot,
                       axis=1, keepdims=True)
        invp_ref[pl.ds(i * C2, C2), :] = slot.astype(jnp.int32)
        return running + jnp.sum(onehot, axis=0, keepdims=True)

    jax.lax.fori_loop(0, nch, rank_body, jnp.zeros((1, E), jnp.float32))

    bs_e = (offsets.astype(jnp.int32) // TB2)
    bc_e = padded_i // TB2
    biota = jax.lax.broadcasted_iota(jnp.int32, (NUM_BLOCKS, E), 0)
    eio_b = jax.lax.broadcasted_iota(jnp.int32, (NUM_BLOCKS, E), 1)
    within = (biota >= bs_e) & (biota < bs_e + bc_e)
    dense = jnp.sum(jnp.where(within, eio_b, 0), axis=1, keepdims=True)
    last_e = jnp.max(jnp.where(bc_e > 0, eio_b[:1, :], -1))
    blk_i = jax.lax.broadcasted_iota(jnp.int32, (NUM_BLOCKS, 1), 0)
    blk_e_ref[...] = jnp.where(blk_i < used, dense, last_e)
    used_ref[...] = jnp.full((1, 1), used, jnp.int32)


def _bookkeep(chosen2d):
    return pl.pallas_call(
        _bookkeep_kernel,
        in_specs=[pl.BlockSpec((T, 1), lambda: (0, 0))],
        out_specs=[
            pl.BlockSpec((T, 1), lambda: (0, 0)),
            pl.BlockSpec((NUM_BLOCKS, 1), lambda: (0, 0)),
            pl.BlockSpec((1, 1), lambda: (0, 0)),
        ],
        out_shape=[
            jax.ShapeDtypeStruct((T, 1), jnp.int32),
            jax.ShapeDtypeStruct((NUM_BLOCKS, 1), jnp.int32),
            jax.ShapeDtypeStruct((1, 1), jnp.int32),
        ],
    )(chosen2d)


@functools.lru_cache(maxsize=None)
def _build_scatter_h():
    mesh = plsc.VectorSubcoreMesh(core_axis_name="c", subcore_axis_name="s")

    @functools.partial(
        pl.kernel,
        mesh=mesh,
        out_type=jax.ShapeDtypeStruct((NUM_SLOTS, HP), jnp.float32),
        scratch_types=[
            pltpu.VMEM((TPW // 128, 128), jnp.int32),
            pltpu.VMEM((TPW, HP), jnp.float32),
            pltpu.SemaphoreType.DMA,
        ],
    )
    def sck(h_hbm, invp_hbm, hs_hbm, idx_v, rows_v, sem):
        wid = jax.lax.axis_index("s") * 2 + jax.lax.axis_index("c")
        base = wid * TPW
        pltpu.sync_copy(h_hbm.at[pl.ds(base, TPW)], rows_v)
        pltpu.sync_copy(invp_hbm.at[pl.ds(wid * (TPW // 128), TPW // 128)], idx_v)
        for j in range(TPW // 128):
            pltpu.async_copy(
                rows_v.at[pl.ds(j * 128, 128)], hs_hbm.at[idx_v.at[j]], sem
            ).wait()

    return sck


def _scatter_h(h, invp2d):
    return _build_scatter_h()(h, invp2d)


def _ffn_kernel(be_ref, us_ref, h_ref, w1_ref, b1_ref, w2_ref, b2_ref, o_ref):
    @pl.when(pl.program_id(0) < us_ref[0])
    def _():
        h = h_ref[...][:, :R].astype(jnp.bfloat16)
        w1 = w1_ref[0].astype(jnp.bfloat16)
        y1 = jax.lax.dot_general(h, w1, (((1,), (1,)), ((), ())),
                                 preferred_element_type=jnp.float32)
        y1 = jnp.maximum(y1 + b1_ref[0], 0.0)
        w2 = w2_ref[0].astype(jnp.bfloat16)
        y = jax.lax.dot_general(y1.astype(jnp.bfloat16), w2,
                                (((1,), (1,)), ((), ())),
                                preferred_element_type=jnp.float32)
        o_ref[...] = y + b2_ref[0]


def _ffn(blk_e, used, hs, W1s, b1s, W2s, b2s):
    grid_spec = pltpu.PrefetchScalarGridSpec(
        num_scalar_prefetch=2,
        grid=(NUM_BLOCKS,),
        in_specs=[
            pl.BlockSpec((TB2, HP), lambda b, be, us: (b, 0)),
            pl.BlockSpec((1, FF, R), lambda b, be, us: (be[b], 0, 0)),
            pl.BlockSpec((1, 1, FF), lambda b, be, us: (be[b], 0, 0)),
            pl.BlockSpec((1, D, FF), lambda b, be, us: (be[b], 0, 0)),
            pl.BlockSpec((1, 1, D), lambda b, be, us: (be[b], 0, 0)),
        ],
        out_specs=pl.BlockSpec(
            (TB2, D), lambda b, be, us: (jnp.minimum(b, us[0] - 1), 0)),
    )
    return pl.pallas_call(
        _ffn_kernel,
        grid_spec=grid_spec,
        out_shape=jax.ShapeDtypeStruct((NUM_SLOTS, D), jnp.float32),
        compiler_params=pltpu.CompilerParams(
            dimension_semantics=("arbitrary",)),
    )(blk_e, used, hs, W1s, b1s, W2s, b2s)


@functools.lru_cache(maxsize=None)
def _build_gather_out():
    mesh = plsc.VectorSubcoreMesh(core_axis_name="c", subcore_axis_name="s")

    @functools.partial(
        pl.kernel,
        mesh=mesh,
        out_type=jax.ShapeDtypeStruct((T, D), jnp.float32),
        scratch_types=[
            pltpu.VMEM((TPW // 128, 128), jnp.int32),
            pltpu.VMEM((128, D), jnp.float32),
            pltpu.SemaphoreType.DMA,
        ],
    )
    def sck(ys_hbm, invp_hbm, out_hbm, idx_v, rows_v, sem):
        wid = jax.lax.axis_index("s") * 2 + jax.lax.axis_index("c")
        base = wid * TPW
        pltpu.sync_copy(invp_hbm.at[pl.ds(wid * (TPW // 128), TPW // 128)], idx_v)
        for j in range(TPW // 128):
            pltpu.async_copy(ys_hbm.at[idx_v.at[j]], rows_v, sem).wait()
            pltpu.sync_copy(rows_v, out_hbm.at[pl.ds(base + j * 128, 128)])

    return sck


def _gather_out(ys, invp2d):
    return _build_gather_out()(ys, invp2d)


def kernel(x, Wg, bg, Ws, bs, W1s, b1s, W2s, b2s):
    G = jax.random.gumbel(jax.random.key(42), (T, E), jnp.float32)
    chosen, h = _gate(x, Wg, bg, G, Ws, bs)
    invp, blk_e, used = _bookkeep(chosen.reshape(T, 1))
    invp2d = invp.reshape(T // 128, 128)
    hs = _scatter_h(h, invp2d)
    ys = _ffn(blk_e.reshape(NUM_BLOCKS), used.reshape(1), hs,
              W1s, b1s.reshape(E, 1, FF), W2s, b2s.reshape(E, 1, D))
    return _gather_out(ys, invp2d)


# speedup vs baseline: 1.0553x; 1.0347x over previous
import functools

import jax
import jax.numpy as jnp
from jax.experimental import pallas as pl
from jax.experimental.pallas import tpu as pltpu
from jax.experimental.pallas import tpu_sc as plsc

T, D, E, R, FF = 8192, 768, 64, 32, 768
TB1 = 2048
TB2 = 256
NUM_BLOCKS = 96
NUM_SLOTS = NUM_BLOCKS * TB2
C2 = 512
NW = 32
TPW = T // NW
HP = 128


NB1 = T // TB1


def _route_kernel(x_ref, wgs_ref, bg_ref, g_ref, bs_ref,
                  h_ref, invp_ref, blk_e_ref, used_ref, ch_ref):
    b = pl.program_id(0)

    @pl.when(b < NB1)
    def _gate_step():
        xb = x_ref[...]
        lgh = jax.lax.dot_general(xb, wgs_ref[...], (((1,), (1,)), ((), ())))
        z = g_ref[...] + (lgh[:, :E] + bg_ref[...])
        chosen = jnp.argmax(z, axis=-1).astype(jnp.int32)
        ch_ref[pl.ds(b * TB1, TB1), :] = chosen[:, None]
        h = jnp.maximum(lgh[:, E:E + R] + bs_ref[...], 0.0)
        h_ref[...] = jnp.concatenate(
            [h, jnp.zeros((TB1, HP - R), jnp.float32)], axis=1)

    @pl.when(b == NB1)
    def _bookkeep_step():
        nch = T // C2
        eio = jax.lax.broadcasted_iota(jnp.int32, (C2, E), 1)

        def count_body(i, counts):
            ch = ch_ref[pl.ds(i * C2, C2), :]
            onehot = (ch == eio).astype(jnp.float32)
            return counts + jnp.sum(onehot, axis=0, keepdims=True)

        counts = jax.lax.fori_loop(
            0, nch, count_body, jnp.zeros((1, E), jnp.float32))

        counts_i = counts.astype(jnp.int32)
        padded_i = ((counts_i + (TB2 - 1)) // TB2) * TB2
        padded_f = padded_i.astype(jnp.float32)

        mtri = (jax.lax.broadcasted_iota(jnp.int32, (E, E), 0)
                < jax.lax.broadcasted_iota(jnp.int32, (E, E), 1)).astype(jnp.float32)
        offsets = jnp.dot(padded_f, mtri, precision=jax.lax.Precision.HIGHEST)

        used = (jnp.sum(padded_i) // TB2).astype(jnp.int32)

        ltri = (jax.lax.broadcasted_iota(jnp.int32, (C2, C2), 1)
                < jax.lax.broadcasted_iota(jnp.int32, (C2, C2), 0)).astype(jnp.float32)

        def rank_body(i, running):
            ch = ch_ref[pl.ds(i * C2, C2), :]
            onehot = (ch == eio).astype(jnp.float32)
            excl = jnp.dot(ltri, onehot)
            slot = jnp.sum((offsets + running + excl) * onehot,
                           axis=1, keepdims=True)
            invp_ref[pl.ds(i * C2, C2), :] = slot.astype(jnp.int32)
            return running + jnp.sum(onehot, axis=0, keepdims=True)

        jax.lax.fori_loop(0, nch, rank_body, jnp.zeros((1, E), jnp.float32))

        bs_e = (offsets.astype(jnp.int32) // TB2)
        bc_e = padded_i // TB2
        biota = jax.lax.broadcasted_iota(jnp.int32, (NUM_BLOCKS, E), 0)
        eio_b = jax.lax.broadcasted_iota(jnp.int32, (NUM_BLOCKS, E), 1)
        within = (biota >= bs_e) & (biota < bs_e + bc_e)
        dense = jnp.sum(jnp.where(within, eio_b, 0), axis=1, keepdims=True)
        last_e = jnp.max(jnp.where(bc_e > 0, eio_b[:1, :], -1))
        blk_i = jax.lax.broadcasted_iota(jnp.int32, (NUM_BLOCKS, 1), 0)
        blk_e_ref[...] = jnp.where(blk_i < used, dense, last_e)
        used_ref[...] = jnp.full((1, 1), used, jnp.int32)


def _route(x, Wg, bg, G, Ws, bs):
    wgs = jnp.concatenate([Wg, Ws], axis=0)
    return pl.pallas_call(
        _route_kernel,
        grid=(NB1 + 1,),
        in_specs=[
            pl.BlockSpec((TB1, D), lambda b: (jnp.minimum(b, NB1 - 1), 0)),
            pl.BlockSpec((E + R, D), lambda b: (0, 0)),
            pl.BlockSpec((E,), lambda b: (0,)),
            pl.BlockSpec((TB1, E), lambda b: (jnp.minimum(b, NB1 - 1), 0)),
            pl.BlockSpec((R,), lambda b: (0,)),
        ],
        out_specs=[
            pl.BlockSpec((TB1, HP), lambda b: (jnp.minimum(b, NB1 - 1), 0)),
            pl.BlockSpec((T, 1), lambda b: (0, 0)),
            pl.BlockSpec((NUM_BLOCKS, 1), lambda b: (0, 0)),
            pl.BlockSpec((1, 1), lambda b: (0, 0)),
        ],
        out_shape=[
            jax.ShapeDtypeStruct((T, HP), jnp.float32),
            jax.ShapeDtypeStruct((T, 1), jnp.int32),
            jax.ShapeDtypeStruct((NUM_BLOCKS, 1), jnp.int32),
            jax.ShapeDtypeStruct((1, 1), jnp.int32),
        ],
        scratch_shapes=[pltpu.VMEM((T, 1), jnp.int32)],
        compiler_params=pltpu.CompilerParams(
            dimension_semantics=("arbitrary",)),
    )(x, wgs, bg, G, bs)


@functools.lru_cache(maxsize=None)
def _build_scatter_h():
    mesh = plsc.VectorSubcoreMesh(core_axis_name="c", subcore_axis_name="s")

    @functools.partial(
        pl.kernel,
        mesh=mesh,
        out_type=jax.ShapeDtypeStruct((NUM_SLOTS, HP), jnp.float32),
        scratch_types=[
            pltpu.VMEM((TPW // 128, 128), jnp.int32),
            pltpu.VMEM((TPW, HP), jnp.float32),
            pltpu.SemaphoreType.DMA,
        ],
    )
    def sck(h_hbm, invp_hbm, hs_hbm, idx_v, rows_v, sem):
        wid = jax.lax.axis_index("s") * 2 + jax.lax.axis_index("c")
        base = wid * TPW
        pltpu.sync_copy(h_hbm.at[pl.ds(base, TPW)], rows_v)
        pltpu.sync_copy(invp_hbm.at[pl.ds(wid * (TPW // 128), TPW // 128)], idx_v)
        for j in range(TPW // 128):
            pltpu.async_copy(
                rows_v.at[pl.ds(j * 128, 128)], hs_hbm.at[idx_v.at[j]], sem
            ).wait()

    return sck


def _scatter_h(h, invp2d):
    return _build_scatter_h()(h, invp2d)


def _ffn_kernel(be_ref, us_ref, h_ref, w1_ref, b1_ref, w2_ref, b2_ref, o_ref):
    @pl.when(pl.program_id(0) < us_ref[0])
    def _():
        h = h_ref[...][:, :R].astype(jnp.bfloat16)
        w1 = w1_ref[0].astype(jnp.bfloat16)
        y1 = jax.lax.dot_general(h, w1, (((1,), (1,)), ((), ())),
                                 preferred_element_type=jnp.float32)
        y1 = jnp.maximum(y1 + b1_ref[0], 0.0)
        w2 = w2_ref[0].astype(jnp.bfloat16)
        y = jax.lax.dot_general(y1.astype(jnp.bfloat16), w2,
                                (((1,), (1,)), ((), ())),
                                preferred_element_type=jnp.float32)
        o_ref[...] = y + b2_ref[0]


def _ffn(blk_e, used, hs, W1s, b1s, W2s, b2s):
    grid_spec = pltpu.PrefetchScalarGridSpec(
        num_scalar_prefetch=2,
        grid=(NUM_BLOCKS,),
        in_specs=[
            pl.BlockSpec((TB2, HP), lambda b, be, us: (b, 0)),
            pl.BlockSpec((1, FF, R), lambda b, be, us: (be[b], 0, 0)),
            pl.BlockSpec((1, 1, FF), lambda b, be, us: (be[b], 0, 0)),
            pl.BlockSpec((1, D, FF), lambda b, be, us: (be[b], 0, 0)),
            pl.BlockSpec((1, 1, D), lambda b, be, us: (be[b], 0, 0)),
        ],
        out_specs=pl.BlockSpec(
            (TB2, D), lambda b, be, us: (jnp.minimum(b, us[0] - 1), 0)),
    )
    return pl.pallas_call(
        _ffn_kernel,
        grid_spec=grid_spec,
        out_shape=jax.ShapeDtypeStruct((NUM_SLOTS, D), jnp.float32),
        compiler_params=pltpu.CompilerParams(
            dimension_semantics=("arbitrary",)),
    )(blk_e, used, hs, W1s, b1s, W2s, b2s)


@functools.lru_cache(maxsize=None)
def _build_gather_out():
    mesh = plsc.VectorSubcoreMesh(core_axis_name="c", subcore_axis_name="s")

    @functools.partial(
        pl.kernel,
        mesh=mesh,
        out_type=jax.ShapeDtypeStruct((T, D), jnp.float32),
        scratch_types=[
            pltpu.VMEM((TPW // 128, 128), jnp.int32),
            pltpu.VMEM((128, D), jnp.float32),
            pltpu.SemaphoreType.DMA,
        ],
    )
    def sck(ys_hbm, invp_hbm, out_hbm, idx_v, rows_v, sem):
        wid = jax.lax.axis_index("s") * 2 + jax.lax.axis_index("c")
        base = wid * TPW
        pltpu.sync_copy(invp_hbm.at[pl.ds(wid * (TPW // 128), TPW // 128)], idx_v)
        for j in range(TPW // 128):
            pltpu.async_copy(ys_hbm.at[idx_v.at[j]], rows_v, sem).wait()
            pltpu.sync_copy(rows_v, out_hbm.at[pl.ds(base + j * 128, 128)])

    return sck


def _gather_out(ys, invp2d):
    return _build_gather_out()(ys, invp2d)


def kernel(x, Wg, bg, Ws, bs, W1s, b1s, W2s, b2s):
    G = jax.random.gumbel(jax.random.key(42), (T, E), jnp.float32)
    h, invp, blk_e, used = _route(x, Wg, bg, G, Ws, bs)
    invp2d = invp.reshape(T // 128, 128)
    hs = _scatter_h(h, invp2d)
    ys = _ffn(blk_e.reshape(NUM_BLOCKS), used.reshape(1), hs,
              W1s, b1s.reshape(E, 1, FF), W2s, b2s.reshape(E, 1, D))
    return _gather_out(ys, invp2d)
